# baseline (speedup 1.0000x reference)
import functools

import jax
import jax.numpy as jnp
from jax import lax
from jax.experimental import pallas as pl
from jax.experimental.pallas import tpu as pltpu
from jax.experimental.pallas import tpu_sc as plsc

_B, _C, _N = 4, 768, 8192
_K = 8
_ROWS = _B * _C
_NW = 32
_RPW = _ROWS // _NW
_LANES = 16
_VPR = _N // _LANES
_BLK = 8
_NB = _VPR // _BLK


def _vsort_desc(v):
    return plsc.sort_key_val(v, v, descending=True)[0]


def _combine(a, b, lane_lt8):
    return _vsort_desc(jnp.where(lane_lt8, a, lax.rev(b, (0,))))


def _insert(ts, x):
    out = []
    cur = x
    for t in ts:
        out.append(jnp.maximum(t, cur))
        cur = jnp.minimum(t, cur)
    return tuple(out)


@functools.partial(
    pl.kernel,
    out_type=jax.ShapeDtypeStruct((_ROWS * _K,), jnp.float32),
    mesh=plsc.VectorSubcoreMesh(core_axis_name="c", subcore_axis_name="s"),
    scratch_types=[
        pltpu.VMEM((_N,), jnp.float32),
        pltpu.VMEM((_N,), jnp.float32),
        pltpu.VMEM((_NB * _LANES,), jnp.float32),
        pltpu.VMEM((_LANES,), jnp.float32),
        pltpu.VMEM((_RPW * _K + _LANES - _K,), jnp.float32),
        pltpu.SemaphoreType.DMA,
        pltpu.SemaphoreType.DMA,
    ],
    compiler_params=pltpu.CompilerParams(needs_layout_passes=False),
)
def _topk_sc(x_hbm, out_hbm, buf0, buf1, bmv, s16, out_v, sem0, sem1):
    nc = 2
    wid = lax.axis_index("s") * nc + lax.axis_index("c")
    base = wid * _RPW
    lane = lax.iota(jnp.int32, 16)
    lane_lt8 = lane < _K
    seven = jnp.full((_LANES,), 7, jnp.int32)
    neg = jnp.full((_LANES,), -jnp.inf, jnp.float32)

    def row_compute(buf, row_local):
        def blk_a(b, m):
            v = [buf[pl.ds(b * _BLK * _LANES + j * _LANES, _LANES)]
                 for j in range(_BLK)]
            m = tuple(jnp.maximum(m[j], jnp.maximum(v[j], v[j + 4]))
                      for j in range(4))
            t01 = jnp.maximum(v[0], v[1])
            t23 = jnp.maximum(v[2], v[3])
            t45 = jnp.maximum(v[4], v[5])
            t67 = jnp.maximum(v[6], v[7])
            bmv[pl.ds(b * _LANES, _LANES)] = jnp.maximum(
                jnp.maximum(t01, t23), jnp.maximum(t45, t67))
            return m

        m = lax.fori_loop(0, _NB, blk_a, (neg,) * 4)
        s = [_vsort_desc(mk) for mk in m]
        cc = _combine(_combine(s[0], s[1], lane_lt8),
                      _combine(s[2], s[3], lane_lt8), lane_lt8)
        s16[...] = cc
        tsplat = plsc.load_gather(s16, [seven])

        def grp(g, ts):
            hits = [bmv[pl.ds((g * 4 + j) * _LANES, _LANES)] >= tsplat
                    for j in range(4)]

            def rescue_group(ts):
                for j in range(4):
                    def rescue_blk(ts, j=j):
                        tsa, tsb = ts[:_K], ts[_K:]
                        off = (g * 4 + j) * _BLK * _LANES
                        for jj in range(_BLK // 2):
                            tsa = _insert(
                                tsa, buf[pl.ds(off + 2 * jj * _LANES, _LANES)])
                            tsb = _insert(
                                tsb, buf[pl.ds(off + (2 * jj + 1) * _LANES,
                                               _LANES)])
                        return tsa + tsb
                    ts = lax.cond(jnp.any(hits[j]), rescue_blk,
                                  lambda ts: ts, ts)
                return ts

            anyhit = jnp.any(hits[0] | hits[1] | hits[2] | hits[3])
            return lax.cond(anyhit, rescue_group, lambda ts: ts, ts)

        ts = lax.fori_loop(0, _NB // 4, grp, (neg,) * (2 * _K))

        tsa, tsb = ts[:_K], ts[_K:]
        vs = [_vsort_desc(jnp.maximum(tsa[i], tsb[_K - 1 - i]))
              for i in range(_K)]
        while len(vs) > 1:
            vs = [_combine(vs[i], vs[i + 1], lane_lt8)
                  for i in range(0, len(vs), 2)]
        plsc.store_compressed(out_v.at[pl.ds(row_local * _K, _LANES)],
                              vs[0], mask=lane_lt8)

    pltpu.async_copy(x_hbm.at[base], buf0, sem0)
    pltpu.async_copy(x_hbm.at[base + 1], buf1, sem1)

    def step(st, carry):
        r0 = 2 * st
        pltpu.make_async_copy(x_hbm.at[base + r0], buf0, sem0).wait()
        row_compute(buf0, r0)
        nxt0 = jnp.minimum(r0 + 2, _RPW - 1)
        pltpu.async_copy(x_hbm.at[base + nxt0], buf0, sem0)

        pltpu.make_async_copy(x_hbm.at[base + r0 + 1], buf1, sem1).wait()
        row_compute(buf1, r0 + 1)
        nxt1 = jnp.minimum(r0 + 3, _RPW - 1)
        pltpu.async_copy(x_hbm.at[base + nxt1], buf1, sem1)
        return carry

    lax.fori_loop(0, _RPW // 2, step, 0)

    pltpu.make_async_copy(x_hbm.at[base], buf0, sem0).wait()
    pltpu.make_async_copy(x_hbm.at[base], buf1, sem1).wait()

    pltpu.sync_copy(out_v.at[pl.ds(0, _RPW * _K)],
                    out_hbm.at[pl.ds(base * _K, _RPW * _K)])


def kernel(x):
    out = _topk_sc(x.reshape(_ROWS, _N))
    return out.reshape(_B, _C, _K)


# speedup vs baseline: 1.9146x; 1.9146x over previous
import functools

import jax
import jax.numpy as jnp
from jax import lax
from jax.experimental import pallas as pl
from jax.experimental.pallas import tpu as pltpu
from jax.experimental.pallas import tpu_sc as plsc

_B, _C, _N = 4, 768, 8192
_K = 8
_ROWS = _B * _C
_NW = 32
_RPW = _ROWS // _NW
_LANES = 16
_VPR = _N // _LANES
_PAIRS = _VPR // 2
_UNROLL = 4
_FB_UNROLL = 4


def _vsort_desc(v):
    return plsc.sort_key_val(v, v, descending=True)[0]


def _combine(a, b, lane_lt8):
    return _vsort_desc(jnp.where(lane_lt8, a, lax.rev(b, (0,))))


def _insert(ts, x):
    out = []
    cur = x
    for t in ts:
        out.append(jnp.maximum(t, cur))
        cur = jnp.minimum(t, cur)
    return tuple(out)


def _merge_tree(vs, lane_lt8):
    vs = [_vsort_desc(t) for t in vs]
    while len(vs) > 1:
        vs = [_combine(vs[i], vs[i + 1], lane_lt8)
              for i in range(0, len(vs), 2)]
    return vs[0]


@functools.partial(
    pl.kernel,
    out_type=jax.ShapeDtypeStruct((_ROWS * _K,), jnp.float32),
    mesh=plsc.VectorSubcoreMesh(core_axis_name="c", subcore_axis_name="s"),
    scratch_types=[
        pltpu.VMEM((_N,), jnp.float32),
        pltpu.VMEM((_N,), jnp.float32),
        pltpu.VMEM((_LANES,), jnp.float32),
        pltpu.VMEM((_RPW * _K + _LANES - _K,), jnp.float32),
        pltpu.SemaphoreType.DMA,
        pltpu.SemaphoreType.DMA,
    ],
    compiler_params=pltpu.CompilerParams(needs_layout_passes=False),
)
def _topk_sc(x_hbm, out_hbm, buf0, buf1, s16, out_v, sem0, sem1):
    nc = 2
    wid = lax.axis_index("s") * nc + lax.axis_index("c")
    base = wid * _RPW
    lane = lax.iota(jnp.int32, 16)
    lane_lt8 = lane < _K
    seven = jnp.full((_LANES,), 7, jnp.int32)
    neg = jnp.full((_LANES,), -jnp.inf, jnp.float32)

    def row_compute(buf, row_local):
        def body(i, carry):
            ts, dmax = carry[:4], carry[4]
            for j in range(_UNROLL):
                off = (i * _UNROLL + j) * 2 * _LANES
                va = buf[pl.ds(off, _LANES)]
                vb = buf[pl.ds(off + _LANES, _LANES)]
                dmax = jnp.maximum(dmax, jnp.minimum(va, vb))
                ts = _insert(ts, jnp.maximum(va, vb))
            return ts + (dmax,)

        carry = lax.fori_loop(0, _PAIRS // _UNROLL, body, (neg,) * 5)
        ts, dmax = carry[:4], carry[4]
        cand = _merge_tree(list(ts), lane_lt8)
        s16[...] = cand
        out8 = plsc.load_gather(s16, [seven])
        viol = jnp.any((ts[3] > out8) | (dmax > out8))

        def fallback():
            def fb_body(i, ts8):
                for j in range(_FB_UNROLL):
                    v = buf[pl.ds((i * _FB_UNROLL + j) * _LANES, _LANES)]
                    ts8 = _insert(ts8, v)
                return ts8
            ts8 = lax.fori_loop(0, _VPR // _FB_UNROLL, fb_body, (neg,) * _K)
            return _merge_tree(list(ts8), lane_lt8)

        final = lax.cond(viol, fallback, lambda: cand)
        plsc.store_compressed(out_v.at[pl.ds(row_local * _K, _LANES)],
                              final, mask=lane_lt8)

    pltpu.async_copy(x_hbm.at[base], buf0, sem0)
    pltpu.async_copy(x_hbm.at[base + 1], buf1, sem1)

    def step(st, carry):
        r0 = 2 * st
        pltpu.make_async_copy(x_hbm.at[base + r0], buf0, sem0).wait()
        row_compute(buf0, r0)
        nxt0 = jnp.minimum(r0 + 2, _RPW - 1)
        pltpu.async_copy(x_hbm.at[base + nxt0], buf0, sem0)

        pltpu.make_async_copy(x_hbm.at[base + r0 + 1], buf1, sem1).wait()
        row_compute(buf1, r0 + 1)
        nxt1 = jnp.minimum(r0 + 3, _RPW - 1)
        pltpu.async_copy(x_hbm.at[base + nxt1], buf1, sem1)
        return carry

    lax.fori_loop(0, _RPW // 2, step, 0)

    pltpu.make_async_copy(x_hbm.at[base], buf0, sem0).wait()
    pltpu.make_async_copy(x_hbm.at[base], buf1, sem1).wait()

    pltpu.sync_copy(out_v.at[pl.ds(0, _RPW * _K)],
                    out_hbm.at[pl.ds(base * _K, _RPW * _K)])


def kernel(x):
    out = _topk_sc(x.reshape(_ROWS, _N))
    return out.reshape(_B, _C, _K)
